# baseline (speedup 1.0000x reference)
import functools

import jax
import jax.numpy as jnp
from jax import lax
from jax.experimental import pallas as pl
from jax.experimental.pallas import tpu as pltpu
from jax.experimental.pallas import tpu_sc as plsc

VOCAB = 1000000
EMBED_DIM = 32
BATCH = 16384
HIST = 50

NC = 2
NS = 16
NW = NC * NS
BT = 128
NBT = BATCH // BT
BT_PER_W = NBT // NW
DT = EMBED_DIM // 8


def _transpose_rows(rows_v, y_v, iota):
    for dt in range(DT):
        for dr in range(8):
            d = dt * 8 + dr
            col = jnp.full((16,), d, jnp.int32)
            for g in range(BT // 16):
                row = iota + (16 * g)
                vec = plsc.load_gather(rows_v, [row, col])
                y_v[dt, dr, pl.ds(16 * g, 16)] = vec


def _body(idx_hbm, tbl_hbm, out_hbm, idxb_v, idxt_v, rows_v, y_v, sem0, sem1):
    wid = lax.axis_index("s") * NC + lax.axis_index("c")
    iota = lax.iota(jnp.int32, 16)
    sems = (sem0, sem1)

    def start(h, slot):
        pltpu.async_copy(tbl_hbm.at[idxt_v.at[h]],
                         rows_v.at[slot], sems[slot])

    def finish(bt, h, slot):
        pltpu.make_async_copy(tbl_hbm.at[pl.ds(0, BT)],
                              rows_v.at[slot], sems[slot]).wait()
        _transpose_rows(rows_v.at[slot], y_v.at[slot], iota)
        pltpu.sync_copy(y_v.at[slot], out_hbm.at[h, :, bt])

    def per_block(i, _):
        bt = wid * BT_PER_W + i
        pltpu.sync_copy(idx_hbm.at[pl.ds(bt * BT, BT)], idxb_v)
        for h in range(HIST):
            col = jnp.full((16,), h, jnp.int32)
            for g in range(BT // 16):
                row = iota + (16 * g)
                idxt_v[h, pl.ds(16 * g, 16)] = plsc.load_gather(
                    idxb_v, [row, col])

        start(0, 0)

        def pair(p, _):
            h = 2 * p
            start(h + 1, 1)
            finish(bt, h, 0)
            start(h + 2, 0)
            finish(bt, h + 1, 1)
            return _

        lax.fori_loop(0, HIST // 2 - 1, pair, None)
        h = HIST - 2
        start(h + 1, 1)
        finish(bt, h, 0)
        finish(bt, h + 1, 1)
        return _

    lax.fori_loop(0, BT_PER_W, per_block, None)


@jax.jit
def _gather(action_ids, table):
    mesh = plsc.VectorSubcoreMesh(core_axis_name="c", subcore_axis_name="s")
    k = functools.partial(
        pl.kernel,
        mesh=mesh,
        out_type=jax.ShapeDtypeStruct((HIST, DT, NBT, 8, BT), jnp.float32),
        scratch_types=[
            pltpu.VMEM((BT, HIST), jnp.int32),
            pltpu.VMEM((HIST, BT), jnp.int32),
            pltpu.VMEM((2, BT, EMBED_DIM), jnp.float32),
            pltpu.VMEM((2, DT, 8, BT), jnp.float32),
            pltpu.SemaphoreType.DMA,
            pltpu.SemaphoreType.DMA,
        ],
        compiler_params=pltpu.CompilerParams(
            use_tc_tiling_on_sc=False, needs_layout_passes=False),
    )(_body)
    out5 = k(action_ids, table)
    t = jnp.transpose(out5, (2, 4, 0, 1, 3))
    return t.reshape(BATCH, HIST, EMBED_DIM)


def kernel(action_ids, table):
    return _gather(action_ids, table)


# speedup vs baseline: 1.1567x; 1.1567x over previous
import functools

import jax
import jax.numpy as jnp
from jax import lax
from jax.experimental import pallas as pl
from jax.experimental.pallas import tpu as pltpu
from jax.experimental.pallas import tpu_sc as plsc

VOCAB = 1000000
EMBED_DIM = 32
BATCH = 16384
HIST = 50

NC = 2
NS = 16
NW = NC * NS
BT = 128
NBT = BATCH // BT
BT_PER_W = NBT // NW
DT = EMBED_DIM // 8


def _transpose_rows(rows_v, y_v, base_idx):
    for b in range(BT):
        for half in range(2):
            vec = rows_v[b, pl.ds(half * 16, 16)]
            plsc.store_scatter(y_v, [base_idx + (half * 16 * BT + b)], vec)


def _body(idxt_hbm, tbl_hbm, out_hbm, idxt_v, rows_v, y_v, sem0, sem1):
    wid = lax.axis_index("s") * NC + lax.axis_index("c")
    base_idx = lax.iota(jnp.int32, 16) * BT
    sems = (sem0, sem1)

    def start(h, slot):
        pltpu.async_copy(tbl_hbm.at[idxt_v.at[h]],
                         rows_v.at[slot], sems[slot])

    def finish(bt, h, slot):
        pltpu.make_async_copy(tbl_hbm.at[pl.ds(0, BT)],
                              rows_v.at[slot], sems[slot]).wait()
        _transpose_rows(rows_v.at[slot], y_v.at[slot], base_idx)
        for dt in range(DT):
            pltpu.sync_copy(y_v.at[slot, pl.ds(dt * 8 * BT, 8 * BT)],
                            out_hbm.at[h, dt, bt])

    def per_block(i, _):
        bt = wid * BT_PER_W + i
        pltpu.sync_copy(idxt_hbm.at[:, pl.ds(bt * BT, BT)], idxt_v)

        start(0, 0)

        def pair(p, _):
            h = 2 * p
            start(h + 1, 1)
            finish(bt, h, 0)
            start(h + 2, 0)
            finish(bt, h + 1, 1)
            return _

        lax.fori_loop(0, HIST // 2 - 1, pair, None)
        h = HIST - 2
        start(h + 1, 1)
        finish(bt, h, 0)
        finish(bt, h + 1, 1)
        return _

    lax.fori_loop(0, BT_PER_W, per_block, None)


@jax.jit
def _gather(action_ids, table):
    idxt = jnp.transpose(action_ids)
    mesh = plsc.VectorSubcoreMesh(core_axis_name="c", subcore_axis_name="s")
    k = functools.partial(
        pl.kernel,
        mesh=mesh,
        out_type=jax.ShapeDtypeStruct((HIST, DT, NBT, 8 * BT), jnp.float32),
        scratch_types=[
            pltpu.VMEM((HIST, BT), jnp.int32),
            pltpu.VMEM((2, BT, EMBED_DIM), jnp.float32),
            pltpu.VMEM((2, DT * 8 * BT), jnp.float32),
            pltpu.SemaphoreType.DMA,
            pltpu.SemaphoreType.DMA,
        ],
        compiler_params=pltpu.CompilerParams(
            use_tc_tiling_on_sc=False, needs_layout_passes=False),
    )(_body)
    out6 = k(idxt, table)
    out5 = out6.reshape(HIST, DT, NBT, 8, BT)
    t = jnp.transpose(out5, (2, 4, 0, 1, 3))
    return t.reshape(BATCH, HIST, EMBED_DIM)


def kernel(action_ids, table):
    return _gather(action_ids, table)


# speedup vs baseline: 1.2198x; 1.0545x over previous
import functools

import jax
import jax.numpy as jnp
from jax import lax
from jax.experimental import pallas as pl
from jax.experimental.pallas import tpu as pltpu
from jax.experimental.pallas import tpu_sc as plsc

VOCAB = 1000000
EMBED_DIM = 32
BATCH = 16384
HIST = 50

NC = 2
NS = 16
NW = NC * NS
BT = 128
NBT = BATCH // BT
BT_PER_W = NBT // NW
DT = EMBED_DIM // 8


def _transpose_rows(rows_v, y_v, base_idx):
    for b0 in range(0, BT, 8):
        vecs = [(b, half, rows_v[b, pl.ds(half * 16, 16)])
                for b in range(b0, b0 + 8) for half in range(2)]
        for b, half, vec in vecs:
            plsc.store_scatter(y_v, [base_idx + (half * 16 * BT + b)], vec)


def _body(idxt_hbm, tbl_hbm, out_hbm, idxt_v, rows_v, y_v, sem0, sem1):
    wid = lax.axis_index("s") * NC + lax.axis_index("c")
    base_idx = lax.iota(jnp.int32, 16) * BT
    sems = (sem0, sem1)

    def start(h, slot):
        pltpu.async_copy(tbl_hbm.at[idxt_v.at[h]],
                         rows_v.at[slot], sems[slot])

    def finish(bt, h, slot):
        pltpu.make_async_copy(tbl_hbm.at[pl.ds(0, BT)],
                              rows_v.at[slot], sems[slot]).wait()
        _transpose_rows(rows_v.at[slot], y_v.at[slot], base_idx)
        for dt in range(DT):
            pltpu.sync_copy(y_v.at[slot, pl.ds(dt * 8 * BT, 8 * BT)],
                            out_hbm.at[h, dt, bt])

    def per_block(i, _):
        bt = wid * BT_PER_W + i
        pltpu.sync_copy(idxt_hbm.at[:, pl.ds(bt * BT, BT)], idxt_v)

        start(0, 0)

        def pair(p, _):
            h = 2 * p
            start(h + 1, 1)
            finish(bt, h, 0)
            start(h + 2, 0)
            finish(bt, h + 1, 1)
            return _

        lax.fori_loop(0, HIST // 2 - 1, pair, None)
        h = HIST - 2
        start(h + 1, 1)
        finish(bt, h, 0)
        finish(bt, h + 1, 1)
        return _

    lax.fori_loop(0, BT_PER_W, per_block, None)


@jax.jit
def _gather(action_ids, table):
    idxt = jnp.transpose(action_ids)
    mesh = plsc.VectorSubcoreMesh(core_axis_name="c", subcore_axis_name="s")
    k = functools.partial(
        pl.kernel,
        mesh=mesh,
        out_type=jax.ShapeDtypeStruct((HIST, DT, NBT, 8 * BT), jnp.float32),
        scratch_types=[
            pltpu.VMEM((HIST, BT), jnp.int32),
            pltpu.VMEM((2, BT, EMBED_DIM), jnp.float32),
            pltpu.VMEM((2, DT * 8 * BT), jnp.float32),
            pltpu.SemaphoreType.DMA,
            pltpu.SemaphoreType.DMA,
        ],
        compiler_params=pltpu.CompilerParams(
            use_tc_tiling_on_sc=False, needs_layout_passes=False),
    )(_body)
    out6 = k(idxt, table)
    out5 = out6.reshape(HIST, DT, NBT, 8, BT)
    t = jnp.transpose(out5, (2, 4, 0, 1, 3))
    return t.reshape(BATCH, HIST, EMBED_DIM)


def kernel(action_ids, table):
    return _gather(action_ids, table)


# speedup vs baseline: 1.5215x; 1.2473x over previous
import functools

import jax
import jax.numpy as jnp
from jax import lax
from jax.experimental import pallas as pl
from jax.experimental.pallas import tpu as pltpu
from jax.experimental.pallas import tpu_sc as plsc

VOCAB = 1000000
EMBED_DIM = 32
BATCH = 16384
HIST = 50

NC = 2
NS = 16
NW = NC * NS
BT = 128
NBT = BATCH // BT
BT_PER_W = NBT // NW
DT = EMBED_DIM // 8


YS = BT + 9


def _transpose_rows(rows_v, y_v, d_idx):
    for b0 in range(0, BT, 8):
        vecs = [(b, half, rows_v[b, pl.ds(half * 16, 16)])
                for b in range(b0, b0 + 8) for half in range(2)]
        for b, half, vec in vecs:
            plsc.store_scatter(y_v, [d_idx[half], jnp.full((16,), b)], vec)


def _body(idxt_hbm, tbl_hbm, out_hbm, idxt_v, rows_v, y_v, sem0, sem1):
    wid = lax.axis_index("s") * NC + lax.axis_index("c")
    iota = lax.iota(jnp.int32, 16)
    d_idx = (iota, iota + 16)
    sems = (sem0, sem1)

    def start(h, slot):
        pltpu.async_copy(tbl_hbm.at[idxt_v.at[h]],
                         rows_v.at[slot], sems[slot])

    def finish(bt, h, slot):
        pltpu.make_async_copy(tbl_hbm.at[pl.ds(0, BT)],
                              rows_v.at[slot], sems[slot]).wait()
        _transpose_rows(rows_v.at[slot], y_v.at[slot], d_idx)
        for dt in range(DT):
            pltpu.sync_copy(y_v.at[slot, pl.ds(dt * 8, 8), pl.ds(0, BT)],
                            out_hbm.at[h, dt, bt])

    def per_block(i, _):
        bt = wid * BT_PER_W + i
        pltpu.sync_copy(idxt_hbm.at[:, pl.ds(bt * BT, BT)], idxt_v)

        start(0, 0)

        def pair(p, _):
            h = 2 * p
            start(h + 1, 1)
            finish(bt, h, 0)
            start(h + 2, 0)
            finish(bt, h + 1, 1)
            return _

        lax.fori_loop(0, HIST // 2 - 1, pair, None)
        h = HIST - 2
        start(h + 1, 1)
        finish(bt, h, 0)
        finish(bt, h + 1, 1)
        return _

    lax.fori_loop(0, BT_PER_W, per_block, None)


@jax.jit
def _gather(action_ids, table):
    idxt = jnp.transpose(action_ids)
    mesh = plsc.VectorSubcoreMesh(core_axis_name="c", subcore_axis_name="s")
    k = functools.partial(
        pl.kernel,
        mesh=mesh,
        out_type=jax.ShapeDtypeStruct((HIST, DT, NBT, 8, BT), jnp.float32),
        scratch_types=[
            pltpu.VMEM((HIST, BT), jnp.int32),
            pltpu.VMEM((2, BT, EMBED_DIM), jnp.float32),
            pltpu.VMEM((2, EMBED_DIM, YS), jnp.float32),
            pltpu.SemaphoreType.DMA,
            pltpu.SemaphoreType.DMA,
        ],
        compiler_params=pltpu.CompilerParams(
            use_tc_tiling_on_sc=False, needs_layout_passes=False),
    )(_body)
    out5 = k(idxt, table)
    t = jnp.transpose(out5, (2, 4, 0, 1, 3))
    return t.reshape(BATCH, HIST, EMBED_DIM)


def kernel(action_ids, table):
    return _gather(action_ids, table)
